# baseline (speedup 1.0000x reference)
import jax
import jax.numpy as jnp
from jax import lax
from jax.experimental import pallas as pl
from jax.experimental.pallas import tpu as pltpu
from jax.experimental.pallas import tpu_sc as plsc

F = 256
N_NODES = 10000
N_EDGES = 160000
NC = 2
NS = 16
NW = NC * NS


_ALPHA = 1.6732632423543772
_SCALE = 1.0507009873554805


def _selu(x):
    return _SCALE * jnp.where(x > 0, x, _ALPHA * (jnp.exp(x) - 1.0))


def _edge_body(e_ref, gu_ref, gw_ref, wa, w2, w3, b1, b2, b3, out_ref):
    x = e_ref[...]
    a1 = jnp.dot(x.astype(jnp.bfloat16), wa[...],
                 preferred_element_type=jnp.float32)
    a1 = a1 + gu_ref[...] + gw_ref[...] + b1[...]
    h1 = _selu(a1)
    a2 = jnp.dot(h1.astype(jnp.bfloat16), w2[...],
                 preferred_element_type=jnp.float32) + b2[...]
    h2 = _selu(a2)
    dm = jnp.dot(h2.astype(jnp.bfloat16), w3[...],
                 preferred_element_type=jnp.float32) + b3[...]
    out_ref[...] = x + dm


def _edge_call(e, gu, gw, wa, w2, w3, b1, b2, b3):
    E = e.shape[0]
    BE = 2000
    grid = (E // BE,)
    row = pl.BlockSpec((BE, F), lambda i: (i, 0))
    full = pl.BlockSpec((F, F), lambda i: (0, 0))
    bias = pl.BlockSpec((1, F), lambda i: (0, 0))
    return pl.pallas_call(
        _edge_body,
        grid=grid,
        in_specs=[row, row, row, full, full, full, bias, bias, bias],
        out_specs=row,
        out_shape=jax.ShapeDtypeStruct((E, F), jnp.float32),
    )(e, gu, gw, wa, w2, w3, b1, b2, b3)


def _node_body_mid(v_ref, s_ref, deg_ref, wna, wnb, w2, w3, b1, b2, b3,
                   wu, ww, v_out, u_out, w_out):
    x = v_ref[...]
    inv = 1.0 / jnp.maximum(deg_ref[...][:, 0:1], 1.0)
    agg = s_ref[...] * inv
    a1 = (jnp.dot(x.astype(jnp.bfloat16), wna[...],
                  preferred_element_type=jnp.float32)
          + jnp.dot(agg.astype(jnp.bfloat16), wnb[...],
                    preferred_element_type=jnp.float32) + b1[...])
    h1 = _selu(a1)
    a2 = jnp.dot(h1.astype(jnp.bfloat16), w2[...],
                 preferred_element_type=jnp.float32) + b2[...]
    h2 = _selu(a2)
    dv = jnp.dot(h2.astype(jnp.bfloat16), w3[...],
                 preferred_element_type=jnp.float32) + b3[...]
    vn = x + dv
    v_out[...] = vn
    vb = vn.astype(jnp.bfloat16)
    u_out[...] = jnp.dot(vb, wu[...], preferred_element_type=jnp.float32)
    w_out[...] = jnp.dot(vb, ww[...], preferred_element_type=jnp.float32)


def _node_body_last(v_ref, s_ref, deg_ref, wna, wnb, w2, w3, b1, b2, b3,
                    v_out):
    x = v_ref[...]
    inv = 1.0 / jnp.maximum(deg_ref[...][:, 0:1], 1.0)
    agg = s_ref[...] * inv
    a1 = (jnp.dot(x.astype(jnp.bfloat16), wna[...],
                  preferred_element_type=jnp.float32)
          + jnp.dot(agg.astype(jnp.bfloat16), wnb[...],
                    preferred_element_type=jnp.float32) + b1[...])
    h1 = _selu(a1)
    a2 = jnp.dot(h1.astype(jnp.bfloat16), w2[...],
                 preferred_element_type=jnp.float32) + b2[...]
    h2 = _selu(a2)
    dv = jnp.dot(h2.astype(jnp.bfloat16), w3[...],
                 preferred_element_type=jnp.float32) + b3[...]
    v_out[...] = x + dv


def _node_call(v, s, deg, wna, wnb, w2, w3, b1, b2, b3, wu=None, ww=None):
    n = v.shape[0]
    BN = 2000
    grid = (n // BN,)
    row = pl.BlockSpec((BN, F), lambda i: (i, 0))
    degs = pl.BlockSpec((BN, deg.shape[1]), lambda i: (i, 0))
    full = pl.BlockSpec((F, F), lambda i: (0, 0))
    bias = pl.BlockSpec((1, F), lambda i: (0, 0))
    if wu is None:
        return pl.pallas_call(
            _node_body_last,
            grid=grid,
            in_specs=[row, row, degs, full, full, full, full,
                      bias, bias, bias],
            out_specs=row,
            out_shape=jax.ShapeDtypeStruct((n, F), jnp.float32),
        )(v, s, deg, wna, wnb, w2, w3, b1, b2, b3)
    return pl.pallas_call(
        _node_body_mid,
        grid=grid,
        in_specs=[row, row, degs, full, full, full, full,
                  bias, bias, bias, full, full],
        out_specs=(row, row, row),
        out_shape=(jax.ShapeDtypeStruct((n, F), jnp.float32),
                   jax.ShapeDtypeStruct((n, F), jnp.float32),
                   jax.ShapeDtypeStruct((n, F), jnp.float32)),
    )(v, s, deg, wna, wnb, w2, w3, b1, b2, b3, wu, ww)


def _pv_body(v_ref, wu, ww, u_out, w_out):
    vb = v_ref[...].astype(jnp.bfloat16)
    u_out[...] = jnp.dot(vb, wu[...], preferred_element_type=jnp.float32)
    w_out[...] = jnp.dot(vb, ww[...], preferred_element_type=jnp.float32)


def _pv_call(v, wu, ww):
    n = v.shape[0]
    BN = 2000
    grid = (n // BN,)
    row = pl.BlockSpec((BN, F), lambda i: (i, 0))
    full = pl.BlockSpec((F, F), lambda i: (0, 0))
    return pl.pallas_call(
        _pv_body,
        grid=grid,
        in_specs=[row, full, full],
        out_specs=(row, row),
        out_shape=(jax.ShapeDtypeStruct((n, F), jnp.float32),
                   jax.ShapeDtypeStruct((n, F), jnp.float32)),
    )(v, wu, ww)


_MESH = plsc.VectorSubcoreMesh(core_axis_name="c", subcore_axis_name="s",
                               num_cores=NC, num_subcores=NS)

_G_CH = 80
_G_TOT = N_EDGES // _G_CH
_G_PW = -(-_G_TOT // NW)


def _gather_body(u_hbm, w_hbm, srcw, dstw, gu_hbm, gw_hbm,
                 idxu, idxw, bu0, bw0, bu1, bw1,
                 g0, g1, ws0, ws1):
    c = lax.axis_index("c")
    s = lax.axis_index("s")
    wid = s * NC + c
    pltpu.sync_copy(srcw.at[wid], idxu)
    pltpu.sync_copy(dstw.at[wid], idxw)

    def fire(j, bu, bw, gsem):
        cid = j * NW + wid

        @pl.when(cid < _G_TOT)
        def _():
            pltpu.async_copy(u_hbm.at[idxu.at[j]], bu, gsem)
            pltpu.async_copy(w_hbm.at[idxw.at[j]], bw, gsem)

    def complete(j, bu, bw, gsem, wsem):
        cid = j * NW + wid

        @pl.when(cid < _G_TOT)
        def _():
            start = cid * _G_CH
            pltpu.make_async_copy(u_hbm.at[idxu.at[j]], bu, gsem).wait()
            pltpu.make_async_copy(w_hbm.at[idxw.at[j]], bw, gsem).wait()
            pltpu.async_copy(bu, gu_hbm.at[pl.ds(start, _G_CH), :], wsem)
            pltpu.async_copy(bw, gw_hbm.at[pl.ds(start, _G_CH), :], wsem)

    def drain(j, bu, bw, wsem):
        cid = j * NW + wid

        @pl.when(cid < _G_TOT)
        def _():
            start = cid * _G_CH
            pltpu.make_async_copy(bu, gu_hbm.at[pl.ds(start, _G_CH), :],
                                  wsem).wait()
            pltpu.make_async_copy(bw, gw_hbm.at[pl.ds(start, _G_CH), :],
                                  wsem).wait()

    fire(0, bu0, bw0, g0)
    fire(1, bu1, bw1, g1)

    def body(jj, carry):
        j0 = 2 * jj
        j1 = j0 + 1
        complete(j0, bu0, bw0, g0, ws0)
        complete(j1, bu1, bw1, g1, ws1)
        drain(j0, bu0, bw0, ws0)
        fire(j0 + 2, bu0, bw0, g0)
        drain(j1, bu1, bw1, ws1)
        fire(j1 + 2, bu1, bw1, g1)
        return carry

    lax.fori_loop(0, (_G_PW + 1) // 2, body, 0)


def _gather_call(u, w, srcw, dstw):
    f = pl.kernel(
        _gather_body,
        out_type=(jax.ShapeDtypeStruct((N_EDGES, F), jnp.float32),
                  jax.ShapeDtypeStruct((N_EDGES, F), jnp.float32)),
        mesh=_MESH,
        scratch_types=[
            pltpu.VMEM((_G_PW, _G_CH), jnp.int32),
            pltpu.VMEM((_G_PW, _G_CH), jnp.int32),
            pltpu.VMEM((_G_CH, F), jnp.float32),
            pltpu.VMEM((_G_CH, F), jnp.float32),
            pltpu.VMEM((_G_CH, F), jnp.float32),
            pltpu.VMEM((_G_CH, F), jnp.float32),
            pltpu.SemaphoreType.DMA,
            pltpu.SemaphoreType.DMA,
            pltpu.SemaphoreType.DMA,
            pltpu.SemaphoreType.DMA,
        ],
    )
    return f(u, w, srcw, dstw)


_A_CH = 80
_A_NCH = (N_EDGES // NS) // _A_CH
_FH = F // NC
_NROW = 624
_NTAIL = N_NODES - NS * _NROW


def _agg_body(e_hbm, dst_hbm, init_hbm, s_hbm, idx_v, b0, b1, b2,
              r0, r1, r2, s0, s1, s2, acc_sh):
    c = lax.axis_index("c")
    s = lax.axis_index("s")
    col0 = c * _FH
    base = s * _NROW
    pltpu.sync_copy(init_hbm.at[pl.ds(base, _NROW), pl.ds(col0, _FH)],
                    acc_sh.at[pl.ds(base, _NROW)])

    @pl.when(s == NS - 1)
    def _():
        pltpu.sync_copy(
            init_hbm.at[pl.ds(NS * _NROW, _NTAIL), pl.ds(col0, _FH)],
            acc_sh.at[pl.ds(NS * _NROW, _NTAIL)])

    pltpu.sync_copy(dst_hbm.at[s], idx_v)
    plsc.subcore_barrier()

    def eslc(j):
        start = s * (_A_NCH * _A_CH) + j * _A_CH
        return e_hbm.at[pl.ds(start, _A_CH), pl.ds(col0, _FH)]

    def fire_read(j, buf, rsem):
        @pl.when(j < _A_NCH)
        def _():
            pltpu.async_copy(eslc(j), buf, rsem)

    def scatter(j, buf, rsem, ssem):
        @pl.when(j < _A_NCH)
        def _():
            pltpu.make_async_copy(eslc(j), buf, rsem).wait()
            pltpu.async_copy(buf, acc_sh.at[idx_v.at[j]], ssem, add=True)

    def drain(j, buf, ssem):
        @pl.when(j < _A_NCH)
        def _():
            pltpu.make_async_copy(buf, acc_sh.at[idx_v.at[j]], ssem).wait()

    fire_read(0, b0, r0)
    fire_read(1, b1, r1)
    fire_read(2, b2, r2)

    def body(jj, carry):
        j0 = 3 * jj
        j1 = j0 + 1
        j2 = j0 + 2
        scatter(j0, b0, r0, s0)
        scatter(j1, b1, r1, s1)
        scatter(j2, b2, r2, s2)
        drain(j0, b0, s0)
        fire_read(j0 + 3, b0, r0)
        drain(j1, b1, s1)
        fire_read(j1 + 3, b1, r1)
        drain(j2, b2, s2)
        fire_read(j2 + 3, b2, r2)
        return carry

    lax.fori_loop(0, -(-_A_NCH // 3), body, 0)
    plsc.subcore_barrier()
    pltpu.sync_copy(acc_sh.at[pl.ds(base, _NROW)],
                    s_hbm.at[pl.ds(base, _NROW), pl.ds(col0, _FH)])

    @pl.when(s == NS - 1)
    def _():
        pltpu.sync_copy(
            acc_sh.at[pl.ds(NS * _NROW, _NTAIL)],
            s_hbm.at[pl.ds(NS * _NROW, _NTAIL), pl.ds(col0, _FH)])


def _agg_call(e_h, dst_h, init):
    f = pl.kernel(
        _agg_body,
        out_type=jax.ShapeDtypeStruct((N_NODES, F), jnp.float32),
        mesh=_MESH,
        scratch_types=[
            pltpu.VMEM((_A_NCH, _A_CH), jnp.int32),
            pltpu.VMEM((_A_CH, _FH), jnp.float32),
            pltpu.VMEM((_A_CH, _FH), jnp.float32),
            pltpu.VMEM((_A_CH, _FH), jnp.float32),
            pltpu.SemaphoreType.DMA,
            pltpu.SemaphoreType.DMA,
            pltpu.SemaphoreType.DMA,
            pltpu.SemaphoreType.DMA,
            pltpu.SemaphoreType.DMA,
            pltpu.SemaphoreType.DMA,
            pltpu.VMEM_SHARED((N_NODES, _FH), jnp.float32),
        ],
    )
    return f(e_h, dst_h, init)


_DW = 128
_D_CH = 80
_D_NCH = (N_EDGES // NS) // _D_CH


def _deg_body(dst_hbm, ones_hbm, zeros_hbm, deg_hbm, idx_v, ones_v,
              dsem, acc_sh):
    c = lax.axis_index("c")
    s = lax.axis_index("s")
    base = s * _NROW
    pltpu.sync_copy(zeros_hbm.at[pl.ds(0, _NROW)],
                    acc_sh.at[pl.ds(base, _NROW)])

    @pl.when(s == NS - 1)
    def _():
        pltpu.sync_copy(zeros_hbm.at[pl.ds(0, _NTAIL)],
                        acc_sh.at[pl.ds(NS * _NROW, _NTAIL)])

    pltpu.sync_copy(dst_hbm.at[s], idx_v)
    pltpu.sync_copy(ones_hbm, ones_v)
    plsc.subcore_barrier()

    def body(j, carry):
        pltpu.async_copy(ones_v, acc_sh.at[idx_v.at[j]], dsem, add=True)
        return carry

    lax.fori_loop(0, _D_NCH, body, 0)

    def drain(j, carry):
        pltpu.make_async_copy(ones_v, acc_sh.at[idx_v.at[j]], dsem).wait()
        return carry

    lax.fori_loop(0, _D_NCH, drain, 0)
    plsc.subcore_barrier()

    @pl.when(c == 0)
    def _():
        pltpu.sync_copy(acc_sh.at[pl.ds(base, _NROW)],
                        deg_hbm.at[pl.ds(base, _NROW), :])

        @pl.when(s == NS - 1)
        def _():
            pltpu.sync_copy(acc_sh.at[pl.ds(NS * _NROW, _NTAIL)],
                            deg_hbm.at[pl.ds(NS * _NROW, _NTAIL), :])


def _deg_call(dst_r, ones128, zeros128):
    f = pl.kernel(
        _deg_body,
        out_type=jax.ShapeDtypeStruct((N_NODES, _DW), jnp.float32),
        mesh=_MESH,
        scratch_types=[
            pltpu.VMEM((_D_NCH, _D_CH), jnp.int32),
            pltpu.VMEM((_D_CH, _DW), jnp.float32),
            pltpu.SemaphoreType.DMA,
            pltpu.VMEM_SHARED((N_NODES, _DW), jnp.float32),
        ],
    )
    return f(dst_r, ones128, zeros128)


def _gidx(x):
    pad = _G_PW * NW * _G_CH - N_EDGES
    xp = jnp.concatenate([x, jnp.zeros((pad,), jnp.int32)])
    return xp.reshape(_G_PW, NW, _G_CH).transpose(1, 0, 2)


def kernel(v, e, edge_index, batch, params):
    del batch
    src = edge_index[0]
    dst = edge_index[1]

    srcw = _gidx(src)
    dstw = _gidx(dst)
    dst_a = dst.reshape(NS, _A_NCH, _A_CH)
    dst_d = dst.reshape(NS, _D_NCH, _D_CH)
    zeros128 = jnp.zeros((_NROW, _FH), jnp.float32)
    ones128 = jnp.ones((_D_CH, _DW), jnp.float32)
    zeros_nodes = jnp.zeros((N_NODES, F), jnp.float32)

    lw = []
    for p in params:
        (w1, b1), (w2, b2), (w3, b3) = p["edge"]
        (n1, c1), (n2, c2), (n3, c3) = p["node"]
        lw.append(dict(
            wa=w1[:F].astype(jnp.bfloat16),
            wu=w1[F:2 * F].astype(jnp.bfloat16),
            ww=w1[2 * F:].astype(jnp.bfloat16),
            w2=w2.astype(jnp.bfloat16), w3=w3.astype(jnp.bfloat16),
            b1=b1.reshape(1, F), b2=b2.reshape(1, F), b3=b3.reshape(1, F),
            wna=n1[:F].astype(jnp.bfloat16), wnb=n1[F:].astype(jnp.bfloat16),
            n2=n2.astype(jnp.bfloat16), n3=n3.astype(jnp.bfloat16),
            c1=c1.reshape(1, F), c2=c2.reshape(1, F), c3=c3.reshape(1, F),
        ))

    deg = _deg_call(dst_d, ones128, zeros128)
    u, w = _pv_call(v, lw[0]["wu"], lw[0]["ww"])
    nl = len(lw)
    for l in range(nl):
        p = lw[l]
        gu, gw = _gather_call(u, w, srcw, dstw)
        e = _edge_call(e, gu, gw, p["wa"], p["w2"], p["w3"],
                       p["b1"], p["b2"], p["b3"])
        s_acc = _agg_call(e, dst_a, zeros_nodes)
        if l + 1 < nl:
            v, u, w = _node_call(v, s_acc, deg, p["wna"], p["wnb"], p["n2"],
                                 p["n3"], p["c1"], p["c2"], p["c3"],
                                 lw[l + 1]["wu"], lw[l + 1]["ww"])
        else:
            v = _node_call(v, s_acc, deg, p["wna"], p["wnb"], p["n2"],
                           p["n3"], p["c1"], p["c2"], p["c3"])
    return (v, e)


# speedup vs baseline: 1.0088x; 1.0088x over previous
import jax
import jax.numpy as jnp
from jax import lax
from jax.experimental import pallas as pl
from jax.experimental.pallas import tpu as pltpu
from jax.experimental.pallas import tpu_sc as plsc

F = 256
N_NODES = 10000
N_EDGES = 160000
NC = 2
NS = 16
NW = NC * NS


_ALPHA = 1.6732632423543772
_SCALE = 1.0507009873554805


def _selu(x):
    return _SCALE * jnp.where(x > 0, x, _ALPHA * (jnp.exp(x) - 1.0))


def _edge_body(e_ref, gu_ref, gw_ref, wa, w2, w3, b1, b2, b3, out_ref):
    x = e_ref[...]
    a1 = jnp.dot(x.astype(jnp.bfloat16), wa[...],
                 preferred_element_type=jnp.float32)
    a1 = a1 + gu_ref[...] + gw_ref[...] + b1[...]
    h1 = _selu(a1)
    a2 = jnp.dot(h1.astype(jnp.bfloat16), w2[...],
                 preferred_element_type=jnp.float32) + b2[...]
    h2 = _selu(a2)
    dm = jnp.dot(h2.astype(jnp.bfloat16), w3[...],
                 preferred_element_type=jnp.float32) + b3[...]
    out_ref[...] = x + dm


def _edge_call(e, gu, gw, wa, w2, w3, b1, b2, b3):
    E = e.shape[0]
    BE = 2000
    grid = (E // BE,)
    row = pl.BlockSpec((BE, F), lambda i: (i, 0))
    full = pl.BlockSpec((F, F), lambda i: (0, 0))
    bias = pl.BlockSpec((1, F), lambda i: (0, 0))
    return pl.pallas_call(
        _edge_body,
        grid=grid,
        in_specs=[row, row, row, full, full, full, bias, bias, bias],
        out_specs=row,
        out_shape=jax.ShapeDtypeStruct((E, F), jnp.float32),
    )(e, gu, gw, wa, w2, w3, b1, b2, b3)


def _node_body_mid(v_ref, s_ref, deg_ref, wna, wnb, w2, w3, b1, b2, b3,
                   wu, ww, v_out, u_out, w_out):
    x = v_ref[...]
    inv = 1.0 / jnp.maximum(deg_ref[...][:, 0:1], 1.0)
    agg = s_ref[...] * inv
    a1 = (jnp.dot(x.astype(jnp.bfloat16), wna[...],
                  preferred_element_type=jnp.float32)
          + jnp.dot(agg.astype(jnp.bfloat16), wnb[...],
                    preferred_element_type=jnp.float32) + b1[...])
    h1 = _selu(a1)
    a2 = jnp.dot(h1.astype(jnp.bfloat16), w2[...],
                 preferred_element_type=jnp.float32) + b2[...]
    h2 = _selu(a2)
    dv = jnp.dot(h2.astype(jnp.bfloat16), w3[...],
                 preferred_element_type=jnp.float32) + b3[...]
    vn = x + dv
    v_out[...] = vn
    vb = vn.astype(jnp.bfloat16)
    u_out[...] = jnp.dot(vb, wu[...], preferred_element_type=jnp.float32)
    w_out[...] = jnp.dot(vb, ww[...], preferred_element_type=jnp.float32)


def _node_body_last(v_ref, s_ref, deg_ref, wna, wnb, w2, w3, b1, b2, b3,
                    v_out):
    x = v_ref[...]
    inv = 1.0 / jnp.maximum(deg_ref[...][:, 0:1], 1.0)
    agg = s_ref[...] * inv
    a1 = (jnp.dot(x.astype(jnp.bfloat16), wna[...],
                  preferred_element_type=jnp.float32)
          + jnp.dot(agg.astype(jnp.bfloat16), wnb[...],
                    preferred_element_type=jnp.float32) + b1[...])
    h1 = _selu(a1)
    a2 = jnp.dot(h1.astype(jnp.bfloat16), w2[...],
                 preferred_element_type=jnp.float32) + b2[...]
    h2 = _selu(a2)
    dv = jnp.dot(h2.astype(jnp.bfloat16), w3[...],
                 preferred_element_type=jnp.float32) + b3[...]
    v_out[...] = x + dv


def _node_call(v, s, deg, wna, wnb, w2, w3, b1, b2, b3, wu=None, ww=None):
    n = v.shape[0]
    BN = 2000
    grid = (n // BN,)
    row = pl.BlockSpec((BN, F), lambda i: (i, 0))
    degs = pl.BlockSpec((BN, deg.shape[1]), lambda i: (i, 0))
    full = pl.BlockSpec((F, F), lambda i: (0, 0))
    bias = pl.BlockSpec((1, F), lambda i: (0, 0))
    if wu is None:
        return pl.pallas_call(
            _node_body_last,
            grid=grid,
            in_specs=[row, row, degs, full, full, full, full,
                      bias, bias, bias],
            out_specs=row,
            out_shape=jax.ShapeDtypeStruct((n, F), jnp.float32),
        )(v, s, deg, wna, wnb, w2, w3, b1, b2, b3)
    return pl.pallas_call(
        _node_body_mid,
        grid=grid,
        in_specs=[row, row, degs, full, full, full, full,
                  bias, bias, bias, full, full],
        out_specs=(row, row, row),
        out_shape=(jax.ShapeDtypeStruct((n, F), jnp.float32),
                   jax.ShapeDtypeStruct((n, F), jnp.float32),
                   jax.ShapeDtypeStruct((n, F), jnp.float32)),
    )(v, s, deg, wna, wnb, w2, w3, b1, b2, b3, wu, ww)


def _pv_body(v_ref, wu, ww, u_out, w_out):
    vb = v_ref[...].astype(jnp.bfloat16)
    u_out[...] = jnp.dot(vb, wu[...], preferred_element_type=jnp.float32)
    w_out[...] = jnp.dot(vb, ww[...], preferred_element_type=jnp.float32)


def _pv_call(v, wu, ww):
    n = v.shape[0]
    BN = 2000
    grid = (n // BN,)
    row = pl.BlockSpec((BN, F), lambda i: (i, 0))
    full = pl.BlockSpec((F, F), lambda i: (0, 0))
    return pl.pallas_call(
        _pv_body,
        grid=grid,
        in_specs=[row, full, full],
        out_specs=(row, row),
        out_shape=(jax.ShapeDtypeStruct((n, F), jnp.float32),
                   jax.ShapeDtypeStruct((n, F), jnp.float32)),
    )(v, wu, ww)


_MESH = plsc.VectorSubcoreMesh(core_axis_name="c", subcore_axis_name="s",
                               num_cores=NC, num_subcores=NS)

_G_CH = 64
_G_TOT = N_EDGES // _G_CH
_G_PW = -(-_G_TOT // NW)


def _gather_body(u_hbm, w_hbm, srcw, dstw, gu_hbm, gw_hbm,
                 idxu, idxw, bu0, bw0, bu1, bw1, bu2, bw2,
                 g0, g1, g2, ws0, ws1, ws2):
    c = lax.axis_index("c")
    s = lax.axis_index("s")
    wid = s * NC + c
    pltpu.sync_copy(srcw.at[wid], idxu)
    pltpu.sync_copy(dstw.at[wid], idxw)

    def fire(j, bu, bw, gsem):
        cid = j * NW + wid

        @pl.when(cid < _G_TOT)
        def _():
            pltpu.async_copy(u_hbm.at[idxu.at[j]], bu, gsem)
            pltpu.async_copy(w_hbm.at[idxw.at[j]], bw, gsem)

    def complete(j, bu, bw, gsem, wsem):
        cid = j * NW + wid

        @pl.when(cid < _G_TOT)
        def _():
            start = cid * _G_CH
            pltpu.make_async_copy(u_hbm.at[idxu.at[j]], bu, gsem).wait()
            pltpu.make_async_copy(w_hbm.at[idxw.at[j]], bw, gsem).wait()
            pltpu.async_copy(bu, gu_hbm.at[pl.ds(start, _G_CH), :], wsem)
            pltpu.async_copy(bw, gw_hbm.at[pl.ds(start, _G_CH), :], wsem)

    def drain(j, bu, bw, wsem):
        cid = j * NW + wid

        @pl.when(cid < _G_TOT)
        def _():
            start = cid * _G_CH
            pltpu.make_async_copy(bu, gu_hbm.at[pl.ds(start, _G_CH), :],
                                  wsem).wait()
            pltpu.make_async_copy(bw, gw_hbm.at[pl.ds(start, _G_CH), :],
                                  wsem).wait()

    fire(0, bu0, bw0, g0)
    fire(1, bu1, bw1, g1)
    fire(2, bu2, bw2, g2)

    def body(jj, carry):
        j0 = 3 * jj
        j1 = j0 + 1
        j2 = j0 + 2
        complete(j0, bu0, bw0, g0, ws0)
        complete(j1, bu1, bw1, g1, ws1)
        complete(j2, bu2, bw2, g2, ws2)
        drain(j0, bu0, bw0, ws0)
        fire(j0 + 3, bu0, bw0, g0)
        drain(j1, bu1, bw1, ws1)
        fire(j1 + 3, bu1, bw1, g1)
        drain(j2, bu2, bw2, ws2)
        fire(j2 + 3, bu2, bw2, g2)
        return carry

    lax.fori_loop(0, -(-_G_PW // 3), body, 0)


def _gather_call(u, w, srcw, dstw):
    f = pl.kernel(
        _gather_body,
        out_type=(jax.ShapeDtypeStruct((N_EDGES, F), jnp.float32),
                  jax.ShapeDtypeStruct((N_EDGES, F), jnp.float32)),
        mesh=_MESH,
        scratch_types=[
            pltpu.VMEM((_G_PW, _G_CH), jnp.int32),
            pltpu.VMEM((_G_PW, _G_CH), jnp.int32),
            pltpu.VMEM((_G_CH, F), jnp.float32),
            pltpu.VMEM((_G_CH, F), jnp.float32),
            pltpu.VMEM((_G_CH, F), jnp.float32),
            pltpu.VMEM((_G_CH, F), jnp.float32),
            pltpu.VMEM((_G_CH, F), jnp.float32),
            pltpu.VMEM((_G_CH, F), jnp.float32),
            pltpu.SemaphoreType.DMA,
            pltpu.SemaphoreType.DMA,
            pltpu.SemaphoreType.DMA,
            pltpu.SemaphoreType.DMA,
            pltpu.SemaphoreType.DMA,
            pltpu.SemaphoreType.DMA,
        ],
    )
    return f(u, w, srcw, dstw)


_A_CH = 80
_A_NCH = (N_EDGES // NS) // _A_CH
_FH = F // NC
_NROW = 624
_NTAIL = N_NODES - NS * _NROW


def _agg_body(e_hbm, dst_hbm, init_hbm, s_hbm, idx_v, b0, b1, b2,
              r0, r1, r2, s0, s1, s2, acc_sh):
    c = lax.axis_index("c")
    s = lax.axis_index("s")
    col0 = c * _FH
    base = s * _NROW
    pltpu.sync_copy(init_hbm.at[pl.ds(base, _NROW), pl.ds(col0, _FH)],
                    acc_sh.at[pl.ds(base, _NROW)])

    @pl.when(s == NS - 1)
    def _():
        pltpu.sync_copy(
            init_hbm.at[pl.ds(NS * _NROW, _NTAIL), pl.ds(col0, _FH)],
            acc_sh.at[pl.ds(NS * _NROW, _NTAIL)])

    pltpu.sync_copy(dst_hbm.at[s], idx_v)
    plsc.subcore_barrier()

    def eslc(j):
        start = s * (_A_NCH * _A_CH) + j * _A_CH
        return e_hbm.at[pl.ds(start, _A_CH), pl.ds(col0, _FH)]

    def fire_read(j, buf, rsem):
        @pl.when(j < _A_NCH)
        def _():
            pltpu.async_copy(eslc(j), buf, rsem)

    def scatter(j, buf, rsem, ssem):
        @pl.when(j < _A_NCH)
        def _():
            pltpu.make_async_copy(eslc(j), buf, rsem).wait()
            pltpu.async_copy(buf, acc_sh.at[idx_v.at[j]], ssem, add=True)

    def drain(j, buf, ssem):
        @pl.when(j < _A_NCH)
        def _():
            pltpu.make_async_copy(buf, acc_sh.at[idx_v.at[j]], ssem).wait()

    fire_read(0, b0, r0)
    fire_read(1, b1, r1)
    fire_read(2, b2, r2)

    def body(jj, carry):
        j0 = 3 * jj
        j1 = j0 + 1
        j2 = j0 + 2
        scatter(j0, b0, r0, s0)
        scatter(j1, b1, r1, s1)
        scatter(j2, b2, r2, s2)
        drain(j0, b0, s0)
        fire_read(j0 + 3, b0, r0)
        drain(j1, b1, s1)
        fire_read(j1 + 3, b1, r1)
        drain(j2, b2, s2)
        fire_read(j2 + 3, b2, r2)
        return carry

    lax.fori_loop(0, -(-_A_NCH // 3), body, 0)
    plsc.subcore_barrier()
    pltpu.sync_copy(acc_sh.at[pl.ds(base, _NROW)],
                    s_hbm.at[pl.ds(base, _NROW), pl.ds(col0, _FH)])

    @pl.when(s == NS - 1)
    def _():
        pltpu.sync_copy(
            acc_sh.at[pl.ds(NS * _NROW, _NTAIL)],
            s_hbm.at[pl.ds(NS * _NROW, _NTAIL), pl.ds(col0, _FH)])


def _agg_call(e_h, dst_h, init):
    f = pl.kernel(
        _agg_body,
        out_type=jax.ShapeDtypeStruct((N_NODES, F), jnp.float32),
        mesh=_MESH,
        scratch_types=[
            pltpu.VMEM((_A_NCH, _A_CH), jnp.int32),
            pltpu.VMEM((_A_CH, _FH), jnp.float32),
            pltpu.VMEM((_A_CH, _FH), jnp.float32),
            pltpu.VMEM((_A_CH, _FH), jnp.float32),
            pltpu.SemaphoreType.DMA,
            pltpu.SemaphoreType.DMA,
            pltpu.SemaphoreType.DMA,
            pltpu.SemaphoreType.DMA,
            pltpu.SemaphoreType.DMA,
            pltpu.SemaphoreType.DMA,
            pltpu.VMEM_SHARED((N_NODES, _FH), jnp.float32),
        ],
    )
    return f(e_h, dst_h, init)


_DW = 128
_D_CH = 80
_D_NCH = (N_EDGES // NS) // _D_CH


def _deg_body(dst_hbm, ones_hbm, zeros_hbm, deg_hbm, idx_v, ones_v,
              dsem, acc_sh):
    c = lax.axis_index("c")
    s = lax.axis_index("s")
    base = s * _NROW
    pltpu.sync_copy(zeros_hbm.at[pl.ds(0, _NROW)],
                    acc_sh.at[pl.ds(base, _NROW)])

    @pl.when(s == NS - 1)
    def _():
        pltpu.sync_copy(zeros_hbm.at[pl.ds(0, _NTAIL)],
                        acc_sh.at[pl.ds(NS * _NROW, _NTAIL)])

    pltpu.sync_copy(dst_hbm.at[s], idx_v)
    pltpu.sync_copy(ones_hbm, ones_v)
    plsc.subcore_barrier()

    def body(j, carry):
        pltpu.async_copy(ones_v, acc_sh.at[idx_v.at[j]], dsem, add=True)
        return carry

    lax.fori_loop(0, _D_NCH, body, 0)

    def drain(j, carry):
        pltpu.make_async_copy(ones_v, acc_sh.at[idx_v.at[j]], dsem).wait()
        return carry

    lax.fori_loop(0, _D_NCH, drain, 0)
    plsc.subcore_barrier()

    @pl.when(c == 0)
    def _():
        pltpu.sync_copy(acc_sh.at[pl.ds(base, _NROW)],
                        deg_hbm.at[pl.ds(base, _NROW), :])

        @pl.when(s == NS - 1)
        def _():
            pltpu.sync_copy(acc_sh.at[pl.ds(NS * _NROW, _NTAIL)],
                            deg_hbm.at[pl.ds(NS * _NROW, _NTAIL), :])


def _deg_call(dst_r, ones128, zeros128):
    f = pl.kernel(
        _deg_body,
        out_type=jax.ShapeDtypeStruct((N_NODES, _DW), jnp.float32),
        mesh=_MESH,
        scratch_types=[
            pltpu.VMEM((_D_NCH, _D_CH), jnp.int32),
            pltpu.VMEM((_D_CH, _DW), jnp.float32),
            pltpu.SemaphoreType.DMA,
            pltpu.VMEM_SHARED((N_NODES, _DW), jnp.float32),
        ],
    )
    return f(dst_r, ones128, zeros128)


def _gidx(x):
    pad = _G_PW * NW * _G_CH - N_EDGES
    xp = jnp.concatenate([x, jnp.zeros((pad,), jnp.int32)])
    return xp.reshape(_G_PW, NW, _G_CH).transpose(1, 0, 2)


def kernel(v, e, edge_index, batch, params):
    del batch
    src = edge_index[0]
    dst = edge_index[1]

    srcw = _gidx(src)
    dstw = _gidx(dst)
    dst_a = dst.reshape(NS, _A_NCH, _A_CH)
    dst_d = dst.reshape(NS, _D_NCH, _D_CH)
    zeros128 = jnp.zeros((_NROW, _FH), jnp.float32)
    ones128 = jnp.ones((_D_CH, _DW), jnp.float32)
    zeros_nodes = jnp.zeros((N_NODES, F), jnp.float32)

    lw = []
    for p in params:
        (w1, b1), (w2, b2), (w3, b3) = p["edge"]
        (n1, c1), (n2, c2), (n3, c3) = p["node"]
        lw.append(dict(
            wa=w1[:F].astype(jnp.bfloat16),
            wu=w1[F:2 * F].astype(jnp.bfloat16),
            ww=w1[2 * F:].astype(jnp.bfloat16),
            w2=w2.astype(jnp.bfloat16), w3=w3.astype(jnp.bfloat16),
            b1=b1.reshape(1, F), b2=b2.reshape(1, F), b3=b3.reshape(1, F),
            wna=n1[:F].astype(jnp.bfloat16), wnb=n1[F:].astype(jnp.bfloat16),
            n2=n2.astype(jnp.bfloat16), n3=n3.astype(jnp.bfloat16),
            c1=c1.reshape(1, F), c2=c2.reshape(1, F), c3=c3.reshape(1, F),
        ))

    deg = _deg_call(dst_d, ones128, zeros128)
    u, w = _pv_call(v, lw[0]["wu"], lw[0]["ww"])
    nl = len(lw)
    for l in range(nl):
        p = lw[l]
        gu, gw = _gather_call(u, w, srcw, dstw)
        e = _edge_call(e, gu, gw, p["wa"], p["w2"], p["w3"],
                       p["b1"], p["b2"], p["b3"])
        s_acc = _agg_call(e, dst_a, zeros_nodes)
        if l + 1 < nl:
            v, u, w = _node_call(v, s_acc, deg, p["wna"], p["wnb"], p["n2"],
                                 p["n3"], p["c1"], p["c2"], p["c3"],
                                 lw[l + 1]["wu"], lw[l + 1]["ww"])
        else:
            v = _node_call(v, s_acc, deg, p["wna"], p["wnb"], p["n2"],
                           p["n3"], p["c1"], p["c2"], p["c3"])
    return (v, e)
